# baseline (speedup 1.0000x reference)
import jax
import jax.numpy as jnp
from jax import lax
from jax.experimental import pallas as pl
from jax.experimental.pallas import tpu as pltpu, tpu_sc as plsc

P = 0.5
ROWS = 64
COLS = 160000
NC, NS = 2, 16
NW = NC * NS
CHUNK = 40000
CHUNKS_PER_ROW = COLS // CHUNK
KA = 2 * CHUNKS_PER_ROW
LANES = 16

_MASK = (1, 0, 0, 0, 1, 0, 1, 0, 0, 0, 1, 0, 0, 0, 1, 0,
         0, 0, 1, 0, 1, 1, 1, 0, 1, 0, 1, 1, 0, 0, 0, 1,
         1, 0, 0, 1, 0, 0, 1, 1, 1, 0, 1, 0, 0, 1, 0, 0,
         0, 1, 0, 1, 1, 0, 0, 1, 1, 1, 0, 0, 1, 1, 0, 1)
def _bits32(bits):
    v = sum(b << i for i, b in enumerate(bits))
    return v - (1 << 32) if v >= (1 << 31) else v


_MLO = _bits32(_MASK[:32])
_MHI = _bits32(_MASK[32:])


def _sc_body(x_hbm, out_hbm, b0, b1, b2, si0, si1, si2, so0, so1, so2):
    bufs = (b0, b1, b2)
    sin = (si0, si1, si2)
    sout = (so0, so1, so2)
    wid = lax.axis_index("s") * NC + lax.axis_index("c")
    base = wid * (2 * COLS)
    mlo = jnp.int32(_MLO)
    mhi = jnp.int32(_MHI)

    def row_flag(p):
        row = 2 * wid + p
        lo_sh = jnp.minimum(row, 31)
        hi_sh = jnp.maximum(row - 32, 0)
        bits = jnp.where(row < 32,
                         lax.shift_right_logical(mlo, lo_sh),
                         lax.shift_right_logical(mhi, hi_sh))
        return (bits & 1) != 0

    negs = [row_flag(0), row_flag(1)]

    def off(j):
        return base + (j // CHUNKS_PER_ROW) * COLS + (j % CHUNKS_PER_ROW) * CHUNK

    def in_start(j):
        pltpu.async_copy(x_hbm.at[pl.ds(off(j), CHUNK)],
                         bufs[j % 3], sin[j % 3])

    def in_wait(j):
        pltpu.make_async_copy(x_hbm.at[pl.ds(off(j), CHUNK)],
                              bufs[j % 3], sin[j % 3]).wait()

    def compute(j):
        buf = bufs[j % 3]

        @pl.when(negs[j // CHUNKS_PER_ROW])
        def _():
            @plsc.parallel_loop(0, CHUNK, LANES, unroll=8)
            def _body(i):
                sl = pl.ds(i, LANES)
                buf[sl] = -buf[sl]

    def out_start(j):
        pltpu.async_copy(bufs[j % 3], out_hbm.at[pl.ds(off(j), CHUNK)],
                         sout[j % 3])

    def out_wait(j):
        pltpu.make_async_copy(bufs[j % 3], out_hbm.at[pl.ds(off(j), CHUNK)],
                              sout[j % 3]).wait()

    in_start(0)
    in_start(1)
    for j in range(KA):
        in_wait(j)
        compute(j)
        out_start(j)
        if j + 2 < KA:
            if j - 1 >= 0:
                out_wait(j - 1)
            in_start(j + 2)
    for j in range(max(KA - 3, 0), KA):
        out_wait(j)


def kernel(x):
    x_flat = x.reshape(-1)
    k = pl.kernel(
        _sc_body,
        out_type=jax.ShapeDtypeStruct((ROWS * COLS,), jnp.float32),
        mesh=plsc.VectorSubcoreMesh(core_axis_name="c", subcore_axis_name="s"),
        scratch_types=[
            pltpu.VMEM((CHUNK,), jnp.float32),
            pltpu.VMEM((CHUNK,), jnp.float32),
            pltpu.VMEM((CHUNK,), jnp.float32),
            pltpu.SemaphoreType.DMA,
            pltpu.SemaphoreType.DMA,
            pltpu.SemaphoreType.DMA,
            pltpu.SemaphoreType.DMA,
            pltpu.SemaphoreType.DMA,
            pltpu.SemaphoreType.DMA,
        ],
    )
    out = k(x_flat)
    return out.reshape(x.shape)
